# speedup vs baseline: 1.1262x; 1.1262x over previous
import functools

import jax
import jax.numpy as jnp
from jax import lax
from jax.experimental import pallas as pl
from jax.experimental.pallas import tpu as pltpu
from jax.experimental.pallas import tpu_sc as plsc

HIDDEN = 768
LANES = 16
VPR = HIDDEN // LANES
NW = 32
C = 24
N_GRAPH = 256
N_NODE = 128
ROWS = N_GRAPH * (N_NODE + 1)
CH_TOTAL = ROWS // C
CH_PER_W = CH_TOTAL // NW
N_SLOT = 11


def _sc_body(idx_hbm, table_hbm, out_hbm, idx_v, acc_v, gbuf_v, sem_g):
    w = lax.axis_index("s") * 2 + lax.axis_index("c")
    ch0 = w * CH_PER_W

    def chunk_body(c, _):
        ch = ch0 + c
        pltpu.sync_copy(idx_hbm.at[ch], idx_v.at[0])
        pltpu.async_copy(table_hbm.at[idx_v.at[0, 0]], acc_v.at[0], sem_g).wait()

        def slot_body(j, _):
            pltpu.async_copy(table_hbm.at[idx_v.at[0, j]], gbuf_v.at[0], sem_g).wait()

            def pos_body(q, _):
                i = q // VPR
                k = q % VPR
                v = gbuf_v[0, i, pl.ds(k * LANES, LANES)]
                plsc.addupdate(acc_v.at[0, i, pl.ds(k * LANES, LANES)], v)
                return _

            lax.fori_loop(0, C * VPR, pos_body, None)
            return _

        lax.fori_loop(1, N_SLOT, slot_body, None)
        pltpu.sync_copy(acc_v.at[0], out_hbm.at[pl.ds(ch * C, C)])
        return _

    lax.fori_loop(0, CH_PER_W, chunk_body, None)


def kernel(x, in_degree, out_degree, atom_table, in_table, out_table, graph_token):
    x = x.astype(jnp.int32)
    in_degree = in_degree.astype(jnp.int32)
    out_degree = out_degree.astype(jnp.int32)
    na = atom_table.shape[0]
    ni = in_table.shape[0]
    token_row = na + ni + out_table.shape[0]

    table = jnp.concatenate([atom_table, in_table, out_table, graph_token], axis=0)

    node_idx = jnp.concatenate(
        [x, (in_degree + na)[..., None], (out_degree + na + ni)[..., None]],
        axis=-1)
    token_idx = jnp.zeros((N_GRAPH, 1, N_SLOT), jnp.int32).at[:, :, 0].set(token_row)
    all_idx = jnp.concatenate([token_idx, node_idx], axis=1).reshape(ROWS, N_SLOT)
    idx = all_idx.reshape(CH_TOTAL, C, N_SLOT).transpose(0, 2, 1)

    mesh = plsc.VectorSubcoreMesh(core_axis_name="c", subcore_axis_name="s")
    out = pl.kernel(
        _sc_body,
        out_type=jax.ShapeDtypeStruct((ROWS, HIDDEN), jnp.float32),
        mesh=mesh,
        scratch_types=[
            pltpu.VMEM((1, N_SLOT, C), jnp.int32),
            pltpu.VMEM((1, C, HIDDEN), jnp.float32),
            pltpu.VMEM((1, C, HIDDEN), jnp.float32),
            pltpu.SemaphoreType.DMA,
        ],
    )(idx, table)
    return out.reshape(N_GRAPH, N_NODE + 1, HIDDEN)
